# Initial kernel scaffold; baseline (speedup 1.0000x reference)
import jax
import jax.numpy as jnp
from jax.experimental import pallas as pl


def kernel(x, W1, b1, W2, b2, W3, b3, Wc, bc, text_feats, Wb, bb, logit_scale):
    raise NotImplementedError("write your pallas kernel here")


# speedup vs baseline: 1.1810x; 1.1810x over previous
import jax
import jax.numpy as jnp
from jax.experimental import pallas as pl
from jax.experimental.pallas import tpu as pltpu

_N, _D, _T, _C = 20000, 1024, 512, 80
_B = 1000


def _dot(a, b):
    return jax.lax.dot_general(a, b, (((1,), (0,)), ((), ())),
                               preferred_element_type=jnp.float32)


def _head_kernel(x_ref, W1_ref, b1_ref, W2_ref, b2_ref, W3_ref, b3_ref,
                 Wc_ref, bc_ref, text_ref, Wb_ref, bb_ref, ls_ref,
                 scores_ref, deltas_ref):
    x = x_ref[...]
    h = _dot(x, W1_ref[...]) + b1_ref[...]
    h = jnp.where(h >= 0, h, 0.01 * h)
    h = _dot(h, W2_ref[...]) + b2_ref[...]
    h = jnp.where(h >= 0, h, 0.01 * h)
    feat = _dot(h, W3_ref[...]) + b3_ref[...]

    emb = _dot(feat, Wc_ref[...]) + bc_ref[...]
    emb = emb / (jnp.sqrt(jnp.sum(emb * emb, axis=-1, keepdims=True)) + 1e-6)
    t = text_ref[...]
    tn = t / (jnp.sqrt(jnp.sum(t * t, axis=-1, keepdims=True)) + 1e-6)
    logits = jax.lax.dot_general(emb, tn, (((1,), (1,)), ((), ())),
                                 preferred_element_type=jnp.float32)
    scores_ref[...] = logits * (1.0 / ls_ref[0, 0])

    deltas_ref[...] = _dot(feat, Wb_ref[...]) + bb_ref[...]


def kernel(x, W1, b1, W2, b2, W3, b3, Wc, bc, text_feats, Wb, bb, logit_scale):
    n, d = x.shape
    t = Wc.shape[1]
    c1 = text_feats.shape[0]
    c4 = Wb.shape[1]
    grid = (n // _B,)

    full = lambda *s: pl.BlockSpec(s, lambda i: (0,) * len(s))
    out_shapes = (
        jax.ShapeDtypeStruct((n, c1), jnp.float32),
        jax.ShapeDtypeStruct((n, c4), jnp.float32),
    )
    scores, deltas = pl.pallas_call(
        _head_kernel,
        grid=grid,
        in_specs=[
            pl.BlockSpec((_B, d), lambda i: (i, 0)),
            full(d, d // 2), full(1, d // 2),
            full(d // 2, d // 2), full(1, d // 2),
            full(d // 2, d), full(1, d),
            full(d, t), full(1, t),
            full(c1, t),
            full(d, c4), full(1, c4),
            full(1, 1),
        ],
        out_specs=(
            pl.BlockSpec((_B, c1), lambda i: (i, 0)),
            pl.BlockSpec((_B, c4), lambda i: (i, 0)),
        ),
        out_shape=out_shapes,
        compiler_params=pltpu.CompilerParams(
            dimension_semantics=("arbitrary",),
        ),
    )(x, W1, b1.reshape(1, -1), W2, b2.reshape(1, -1), W3, b3.reshape(1, -1),
      Wc, bc.reshape(1, -1), text_feats, Wb, bb.reshape(1, -1),
      jnp.asarray(logit_scale, jnp.float32).reshape(1, 1))
    return scores, deltas
